# Initial kernel scaffold; baseline (speedup 1.0000x reference)
import jax
import jax.numpy as jnp
from jax.experimental import pallas as pl


def kernel(x, edge_index, edge_attr, batch, params):
    raise NotImplementedError("write your pallas kernel here")


# speedup vs baseline: 2.5646x; 2.5646x over previous
import functools

import jax
import jax.numpy as jnp
from jax import lax
from jax.experimental import pallas as pl
from jax.experimental.pallas import tpu as pltpu
from jax.experimental.pallas import tpu_sc as plsc

H = 128
_F32 = jnp.float32


def _mlp_sum(inputs, w1s, b1, w2, b2, posts=(), post_adds=None, emit_y=True,
             block_rows=1000):
    nin, npost = len(inputs), len(posts)
    if post_adds is None:
        post_adds = [None] * npost
    adds = [a for a in post_adds if a is not None]
    has_add = [a is not None for a in post_adds]
    rows = inputs[0].shape[0]
    assert rows % block_rows == 0
    grid = rows // block_rows

    def body(*refs):
        ins = refs[:nin]
        w1r = refs[nin:2 * nin]
        b1r, w2r, b2r = refs[2 * nin:2 * nin + 3]
        pw = refs[2 * nin + 3:2 * nin + 3 + npost]
        ad = refs[2 * nin + 3 + npost:2 * nin + 3 + npost + len(adds)]
        outs = refs[2 * nin + 3 + npost + len(adds):]
        acc = b1r[...].astype(_F32)
        for k in range(nin):
            acc = acc + jnp.dot(ins[k][...], w1r[k][...],
                                preferred_element_type=_F32)
        hid = jnp.maximum(acc, 0.0)
        y = jnp.dot(hid, w2r[...], preferred_element_type=_F32) + b2r[...]
        oi = 0
        if emit_y:
            outs[0][...] = y
            oi = 1
        ai = 0
        for j in range(npost):
            pv = jnp.dot(y, pw[j][...], preferred_element_type=_F32)
            if has_add[j]:
                pv = pv + ad[ai][...]
                ai += 1
            outs[oi + j][...] = pv

    in_specs = []
    for a in inputs:
        d = a.shape[1]
        in_specs.append(pl.BlockSpec((block_rows, d), lambda i: (i, 0)))
    for w in w1s:
        in_specs.append(pl.BlockSpec(w.shape, lambda i: (0, 0)))
    in_specs.append(pl.BlockSpec((1, H), lambda i: (0, 0)))
    in_specs.append(pl.BlockSpec(w2.shape, lambda i: (0, 0)))
    in_specs.append(pl.BlockSpec((1, H), lambda i: (0, 0)))
    for p in posts:
        in_specs.append(pl.BlockSpec(p.shape, lambda i: (0, 0)))
    for a in adds:
        in_specs.append(pl.BlockSpec((block_rows, H), lambda i: (i, 0)))

    n_out = (1 if emit_y else 0) + npost
    out_shape = [jax.ShapeDtypeStruct((rows, H), _F32) for _ in range(n_out)]
    out_specs = [pl.BlockSpec((block_rows, H), lambda i: (i, 0))
                 for _ in range(n_out)]

    fn = pl.pallas_call(
        body,
        grid=(grid,),
        in_specs=in_specs,
        out_specs=out_specs,
        out_shape=out_shape,
    )
    args = (list(inputs) + list(w1s)
            + [b1.reshape(1, H), w2, b2.reshape(1, H)] + list(posts) + adds)
    return tuple(fn(*args))


def _edge_mlp(ug, vg, t, b1, w2, b2, posts=(), block_rows=2000):
    npost = len(posts)
    rows = ug.shape[0]
    assert rows % block_rows == 0
    grid = rows // block_rows

    def body(*refs):
        ugr, vgr, tr, b1r, w2r, b2r = refs[:6]
        pw = refs[6:6 + npost]
        outs = refs[6 + npost:]
        s = ugr[...] + vgr[...] + tr[...] + b1r[...]
        hid = jnp.maximum(s, 0.0)
        ea = jnp.dot(hid, w2r[...], preferred_element_type=_F32) + b2r[...]
        outs[0][...] = ea
        for j in range(npost):
            outs[1 + j][...] = jnp.dot(ea, pw[j][...],
                                       preferred_element_type=_F32)

    in_specs = [pl.BlockSpec((block_rows, H), lambda i: (i, 0)),
                pl.BlockSpec((block_rows, H), lambda i: (i, 0)),
                pl.BlockSpec((block_rows, H), lambda i: (i, 0)),
                pl.BlockSpec((1, H), lambda i: (0, 0)),
                pl.BlockSpec((H, H), lambda i: (0, 0)),
                pl.BlockSpec((1, H), lambda i: (0, 0))]
    for p in posts:
        in_specs.append(pl.BlockSpec(p.shape, lambda i: (0, 0)))
    out_shape = [jax.ShapeDtypeStruct((rows, H), _F32)
                 for _ in range(1 + npost)]
    out_specs = [pl.BlockSpec((block_rows, H), lambda i: (i, 0))
                 for _ in range(1 + npost)]
    fn = pl.pallas_call(body, grid=(grid,), in_specs=in_specs,
                        out_specs=out_specs, out_shape=out_shape)
    return tuple(fn(ug, vg, t, b1.reshape(1, H), w2, b2.reshape(1, H), *posts))


_NC, _NS = 2, 16
_NW = _NC * _NS
_C = 128


def _sc_gather(u, v, row, col):
    e = row.shape[0]
    assert e % _NW == 0
    epw = e // _NW
    nfull, tail = divmod(epw, _C)
    assert epw % 8 == 0 and (_C % 8 == 0)
    mesh = plsc.VectorSubcoreMesh(core_axis_name="c", subcore_axis_name="s")
    scratch = [pltpu.VMEM((_C,), jnp.int32), pltpu.VMEM((_C,), jnp.int32),
               pltpu.VMEM((_C, H), _F32), pltpu.VMEM((_C, H), _F32)]
    if tail:
        scratch += [pltpu.VMEM((tail,), jnp.int32),
                    pltpu.VMEM((tail,), jnp.int32),
                    pltpu.VMEM((tail, H), _F32), pltpu.VMEM((tail, H), _F32)]
    scratch.append(pltpu.SemaphoreType.DMA)

    def body(u_hbm, v_hbm, row_hbm, col_hbm, ug_hbm, vg_hbm, *scr):
        if tail:
            idr, idc, bu, bv, idrt, idct, but, bvt, sem = scr
        else:
            idr, idc, bu, bv, sem = scr
        w = lax.axis_index("c") * _NS + lax.axis_index("s")
        base = w * epw

        def chunk(off, n, ir, ic, xu, xv):
            pltpu.sync_copy(row_hbm.at[pl.ds(off, n)], ir)
            pltpu.sync_copy(col_hbm.at[pl.ds(off, n)], ic)
            pltpu.async_copy(u_hbm.at[ir], xu, sem).wait()
            pltpu.async_copy(v_hbm.at[ic], xv, sem).wait()
            pltpu.sync_copy(xu, ug_hbm.at[pl.ds(off, n)])
            pltpu.sync_copy(xv, vg_hbm.at[pl.ds(off, n)])

        def step(i, carry):
            chunk(base + i * _C, _C, idr, idc, bu, bv)
            return carry

        lax.fori_loop(0, nfull, step, 0)
        if tail:
            chunk(base + nfull * _C, tail, idrt, idct, but, bvt)

    k = pl.kernel(
        body,
        out_type=(jax.ShapeDtypeStruct((e, H), _F32),
                  jax.ShapeDtypeStruct((e, H), _F32)),
        mesh=mesh,
        scratch_types=scratch,
    )
    return k(u, v, row, col)


def _scatter_pad(n):
    rpt = ((n + _NS * 8 - 1) // (_NS * 8)) * 8
    return rpt, _NS * rpt


def _sc_scatter(ea, row, zeros, n):
    e = row.shape[0]
    rpt, npad = _scatter_pad(n)
    assert e % _NC == 0
    epc = e // _NC
    assert epc % _NS == 0
    ept = epc // _NS
    nfull, tail = divmod(ept, _C)
    mesh = plsc.VectorSubcoreMesh(core_axis_name="c", subcore_axis_name="s")
    scratch = [pltpu.VMEM_SHARED((npad, H), _F32),
               pltpu.VMEM((_C,), jnp.int32), pltpu.VMEM((_C, H), _F32)]
    if tail:
        scratch += [pltpu.VMEM((tail,), jnp.int32),
                    pltpu.VMEM((tail, H), _F32)]
    scratch.append(pltpu.SemaphoreType.DMA)

    def body(ea_hbm, row_hbm, z_hbm, out_hbm, *scr):
        if tail:
            acc, idx, buf, idxt, buft, sem = scr
        else:
            acc, idx, buf, sem = scr
        c = lax.axis_index("c")
        s = lax.axis_index("s")
        pltpu.sync_copy(z_hbm, acc.at[pl.ds(s * rpt, rpt)])
        plsc.subcore_barrier()
        base = c * epc + s * ept

        def chunk(off, n_, ix, bf):
            pltpu.sync_copy(row_hbm.at[pl.ds(off, n_)], ix)
            pltpu.sync_copy(ea_hbm.at[pl.ds(off, n_)], bf)
            pltpu.sync_copy(bf, acc.at[ix], add=True)

        def step(i, carry):
            chunk(base + i * _C, _C, idx, buf)
            return carry

        lax.fori_loop(0, nfull, step, 0)
        if tail:
            chunk(base + nfull * _C, tail, idxt, buft)
        plsc.subcore_barrier()
        pltpu.sync_copy(acc.at[pl.ds(s * rpt, rpt)],
                        out_hbm.at[pl.ds(c * npad + s * rpt, rpt)])

    k = pl.kernel(
        body,
        out_type=jax.ShapeDtypeStruct((_NC * npad, H), _F32),
        mesh=mesh,
        scratch_types=scratch,
    )
    return k(ea, row, zeros)


def kernel(x, edge_index, edge_attr, batch, params):
    n = x.shape[0]
    row, col = edge_index[0], edge_index[1]
    rpt, npad = _scatter_pad(n)
    zeros = jnp.zeros((rpt, H), _F32)

    ne_w1, ne_b1, ne_w2, ne_b2 = params["ne"]
    ee_w1, ee_b1, ee_w2, ee_b2 = params["ee"]
    nd_w1, nd_b1, nd_w2, nd_b2 = params["nd"]
    e0w1, e0b1, e0w2, e0b2 = params["enc0_edge"]
    e1w1, e1b1, e1w2, e1b2 = params["enc1_edge"]
    n0w1, n0b1, n0w2, n0b2 = params["enc0_node"]
    n1w1, n1b1, n1w2, n1b2 = params["enc1_node"]
    d0ew1, d0eb1, d0ew2, d0eb2 = params["dec0_edge"]
    d1ew1, d1eb1, d1ew2, d1eb2 = params["dec1_edge"]
    d0nw1, d0nb1, d0nw2, d0nb2 = params["dec0_node"]
    d1nw1, d1nb1, d1nw2, d1nb2 = params["dec1_node"]

    e0_r, e0_c, e0_e = e0w1[:H], e0w1[H:2 * H], e0w1[2 * H:]
    e1_r, e1_c, e1_e = e1w1[:H], e1w1[H:2 * H], e1w1[2 * H:]
    n0_x, n0_a = n0w1[:H], n0w1[H:]
    n1_x, n1_a = n1w1[:H], n1w1[H:]
    d0_xr, d0_er, d0_xc, d0_ec, d0_e = (d0ew1[:H], d0ew1[H:2 * H],
                                        d0ew1[2 * H:3 * H],
                                        d0ew1[3 * H:4 * H], d0ew1[4 * H:])
    d1_xr, d1_er, d1_xc, d1_ec, d1_e = (d1ew1[:H], d1ew1[H:2 * H],
                                        d1ew1[2 * H:3 * H],
                                        d1ew1[3 * H:4 * H], d1ew1[4 * H:])
    d0_nx, d0_ne, d0_na = d0nw1[:H], d0nw1[H:2 * H], d0nw1[2 * H:]
    d1_nx, d1_ne, d1_na = d1nw1[:H], d1nw1[H:2 * H], d1nw1[2 * H:]

    x1, u0, v0 = _mlp_sum([x], [ne_w1], ne_b1, ne_w2, ne_b2,
                          posts=[e0_r, e0_c])
    (t0,) = _mlp_sum([edge_attr], [ee_w1], ee_b1, ee_w2, ee_b2,
                     posts=[e0_e], emit_y=False, block_rows=2000)

    ug, vg = _sc_gather(u0, v0, row, col)
    ea_e0, t_e1, t_d1 = _edge_mlp(ug, vg, t0, e0b1, e0w2, e0b2,
                                  posts=[e1_e, d1_e])
    parts = _sc_scatter(ea_e0, row, zeros, n)
    p0, p1 = parts[:npad], parts[npad:]
    x2, u1, v1, ud1p, vd1p = _mlp_sum(
        [x1, p0, p1], [n0_x, n0_a, n0_a], n0b1, n0w2, n0b2,
        posts=[e1_r, e1_c, d1_er, d1_ec])

    ug, vg = _sc_gather(u1, v1, row, col)
    ea_e1, t_d0 = _edge_mlp(ug, vg, t_e1, e1b1, e1w2, e1b2, posts=[d0_e])
    parts = _sc_scatter(ea_e1, row, zeros, n)
    p0, p1 = parts[:npad], parts[npad:]
    x3, ud0, vd0 = _mlp_sum(
        [x2, p0, p1], [n1_x, n1_a, n1_a], n1b1, n1w2, n1b2,
        posts=[d0_xr + d0_er, d0_xc + d0_ec])

    ug, vg = _sc_gather(ud0, vd0, row, col)
    (ea_d0,) = _edge_mlp(ug, vg, t_d0, d0eb1, d0ew2, d0eb2)
    parts = _sc_scatter(ea_d0, row, zeros, n)
    p0, p1 = parts[:npad], parts[npad:]
    x4, ud1, vd1 = _mlp_sum(
        [x3, p0, p1], [d0_nx + d0_ne, d0_na, d0_na], d0nb1, d0nw2, d0nb2,
        posts=[d1_xr, d1_xc], post_adds=[ud1p, vd1p])

    ug, vg = _sc_gather(ud1, vd1, row, col)
    (ea_d1,) = _edge_mlp(ug, vg, t_d1, d1eb1, d1ew2, d1eb2)
    parts = _sc_scatter(ea_d1, row, zeros, n)
    p0, p1 = parts[:npad], parts[npad:]
    (x5,) = _mlp_sum([x4, x2, p0, p1], [d1_nx, d1_ne, d1_na, d1_na],
                     d1nb1, d1nw2, d1nb2)

    (out,) = _mlp_sum([x5], [nd_w1], nd_b1, nd_w2, nd_b2)
    return out


# speedup vs baseline: 3.4175x; 1.3326x over previous
import functools

import jax
import jax.numpy as jnp
from jax import lax
from jax.experimental import pallas as pl
from jax.experimental.pallas import tpu as pltpu
from jax.experimental.pallas import tpu_sc as plsc

H = 128
_F32 = jnp.float32


def _mlp_sum(inputs, w1s, b1, w2, b2, posts=(), post_adds=None, emit_y=True,
             block_rows=1000):
    nin, npost = len(inputs), len(posts)
    if post_adds is None:
        post_adds = [None] * npost
    adds = [a for a in post_adds if a is not None]
    has_add = [a is not None for a in post_adds]
    rows = inputs[0].shape[0]
    assert rows % block_rows == 0
    grid = rows // block_rows

    def body(*refs):
        ins = refs[:nin]
        w1r = refs[nin:2 * nin]
        b1r, w2r, b2r = refs[2 * nin:2 * nin + 3]
        pw = refs[2 * nin + 3:2 * nin + 3 + npost]
        ad = refs[2 * nin + 3 + npost:2 * nin + 3 + npost + len(adds)]
        outs = refs[2 * nin + 3 + npost + len(adds):]
        acc = b1r[...].astype(_F32)
        for k in range(nin):
            acc = acc + jnp.dot(ins[k][...], w1r[k][...],
                                preferred_element_type=_F32)
        hid = jnp.maximum(acc, 0.0)
        y = jnp.dot(hid, w2r[...], preferred_element_type=_F32) + b2r[...]
        oi = 0
        if emit_y:
            outs[0][...] = y
            oi = 1
        ai = 0
        for j in range(npost):
            pv = jnp.dot(y, pw[j][...], preferred_element_type=_F32)
            if has_add[j]:
                pv = pv + ad[ai][...]
                ai += 1
            outs[oi + j][...] = pv

    in_specs = []
    for a in inputs:
        d = a.shape[1]
        in_specs.append(pl.BlockSpec((block_rows, d), lambda i: (i, 0)))
    for w in w1s:
        in_specs.append(pl.BlockSpec(w.shape, lambda i: (0, 0)))
    in_specs.append(pl.BlockSpec((1, H), lambda i: (0, 0)))
    in_specs.append(pl.BlockSpec(w2.shape, lambda i: (0, 0)))
    in_specs.append(pl.BlockSpec((1, H), lambda i: (0, 0)))
    for p in posts:
        in_specs.append(pl.BlockSpec(p.shape, lambda i: (0, 0)))
    for a in adds:
        in_specs.append(pl.BlockSpec((block_rows, H), lambda i: (i, 0)))

    n_out = (1 if emit_y else 0) + npost
    out_shape = [jax.ShapeDtypeStruct((rows, H), _F32) for _ in range(n_out)]
    out_specs = [pl.BlockSpec((block_rows, H), lambda i: (i, 0))
                 for _ in range(n_out)]

    fn = pl.pallas_call(
        body,
        grid=(grid,),
        in_specs=in_specs,
        out_specs=out_specs,
        out_shape=out_shape,
    )
    args = (list(inputs) + list(w1s)
            + [b1.reshape(1, H), w2, b2.reshape(1, H)] + list(posts) + adds)
    return tuple(fn(*args))


def _edge_mlp(ug, vg, t, b1, w2, b2, posts=(), block_rows=2000):
    npost = len(posts)
    rows = ug.shape[0]
    assert rows % block_rows == 0
    grid = rows // block_rows

    def body(*refs):
        ugr, vgr, tr, b1r, w2r, b2r = refs[:6]
        pw = refs[6:6 + npost]
        outs = refs[6 + npost:]
        s = ugr[...] + vgr[...] + tr[...] + b1r[...]
        hid = jnp.maximum(s, 0.0)
        ea = jnp.dot(hid, w2r[...], preferred_element_type=_F32) + b2r[...]
        outs[0][...] = ea
        for j in range(npost):
            outs[1 + j][...] = jnp.dot(ea, pw[j][...],
                                       preferred_element_type=_F32)

    in_specs = [pl.BlockSpec((block_rows, H), lambda i: (i, 0)),
                pl.BlockSpec((block_rows, H), lambda i: (i, 0)),
                pl.BlockSpec((block_rows, H), lambda i: (i, 0)),
                pl.BlockSpec((1, H), lambda i: (0, 0)),
                pl.BlockSpec((H, H), lambda i: (0, 0)),
                pl.BlockSpec((1, H), lambda i: (0, 0))]
    for p in posts:
        in_specs.append(pl.BlockSpec(p.shape, lambda i: (0, 0)))
    out_shape = [jax.ShapeDtypeStruct((rows, H), _F32)
                 for _ in range(1 + npost)]
    out_specs = [pl.BlockSpec((block_rows, H), lambda i: (i, 0))
                 for _ in range(1 + npost)]
    fn = pl.pallas_call(body, grid=(grid,), in_specs=in_specs,
                        out_specs=out_specs, out_shape=out_shape)
    return tuple(fn(ug, vg, t, b1.reshape(1, H), w2, b2.reshape(1, H), *posts))


_NC, _NS = 2, 16
_NW = _NC * _NS
_C = 80
_D = 5


def _sc_gather(u, v, row, col):
    e = row.shape[0]
    assert e % (_NW * _C * _D) == 0
    epw = e // _NW
    groups = epw // (_C * _D)
    mesh = plsc.VectorSubcoreMesh(core_axis_name="c", subcore_axis_name="s")
    scratch = ([pltpu.VMEM((_C,), jnp.int32) for _ in range(2 * _D)]
               + [pltpu.VMEM((_C, H), _F32) for _ in range(2 * _D)]
               + [pltpu.SemaphoreType.DMA] * 3)

    def body(u_hbm, v_hbm, row_hbm, col_hbm, ug_hbm, vg_hbm, *scr):
        idr = scr[0:_D]
        idc = scr[_D:2 * _D]
        bu = scr[2 * _D:3 * _D]
        bv = scr[3 * _D:4 * _D]
        semi, semg, semw = scr[4 * _D:]
        w = lax.axis_index("c") * _NS + lax.axis_index("s")
        base = w * epw

        def group(g, carry):
            off0 = base + g * (_C * _D)
            di = [(pltpu.async_copy(row_hbm.at[pl.ds(off0 + b * _C, _C)],
                                    idr[b], semi),
                   pltpu.async_copy(col_hbm.at[pl.ds(off0 + b * _C, _C)],
                                    idc[b], semi)) for b in range(_D)]
            dg = []
            for b in range(_D):
                di[b][0].wait()
                di[b][1].wait()
                dg.append((pltpu.async_copy(u_hbm.at[idr[b]], bu[b], semg),
                           pltpu.async_copy(v_hbm.at[idc[b]], bv[b], semg)))
            dw = []
            for b in range(_D):
                dg[b][0].wait()
                dg[b][1].wait()
                off = off0 + b * _C
                dw.append((pltpu.async_copy(bu[b], ug_hbm.at[pl.ds(off, _C)],
                                            semw),
                           pltpu.async_copy(bv[b], vg_hbm.at[pl.ds(off, _C)],
                                            semw)))
            for b in range(_D):
                dw[b][0].wait()
                dw[b][1].wait()
            return carry

        lax.fori_loop(0, groups, group, 0)

    k = pl.kernel(
        body,
        out_type=(jax.ShapeDtypeStruct((e, H), _F32),
                  jax.ShapeDtypeStruct((e, H), _F32)),
        mesh=mesh,
        scratch_types=scratch,
    )
    return k(u, v, row, col)


def _scatter_pad(n):
    rpt = ((n + _NS * 8 - 1) // (_NS * 8)) * 8
    return rpt, _NS * rpt


def _sc_scatter(ea, row, zeros, n):
    e = row.shape[0]
    rpt, npad = _scatter_pad(n)
    cs, ds = 40, 5
    assert e % (_NC * _NS * cs * ds) == 0
    epc = e // _NC
    ept = epc // _NS
    groups = ept // (cs * ds)
    mesh = plsc.VectorSubcoreMesh(core_axis_name="c", subcore_axis_name="s")
    scratch = ([pltpu.VMEM_SHARED((npad, H), _F32)]
               + [pltpu.VMEM((cs,), jnp.int32) for _ in range(ds)]
               + [pltpu.VMEM((cs, H), _F32) for _ in range(ds)]
               + [pltpu.SemaphoreType.DMA] * 2)

    def body(ea_hbm, row_hbm, z_hbm, out_hbm, *scr):
        acc = scr[0]
        idx = scr[1:1 + ds]
        buf = scr[1 + ds:1 + 2 * ds]
        semi, sema = scr[1 + 2 * ds:]
        c = lax.axis_index("c")
        s = lax.axis_index("s")
        pltpu.sync_copy(z_hbm, acc.at[pl.ds(s * rpt, rpt)])
        plsc.subcore_barrier()
        base = c * epc + s * ept

        def group(g, carry):
            off0 = base + g * (cs * ds)
            di = [(pltpu.async_copy(row_hbm.at[pl.ds(off0 + b * cs, cs)],
                                    idx[b], semi),
                   pltpu.async_copy(ea_hbm.at[pl.ds(off0 + b * cs, cs)],
                                    buf[b], semi)) for b in range(ds)]
            da = []
            for b in range(ds):
                di[b][0].wait()
                di[b][1].wait()
                da.append(pltpu.async_copy(buf[b], acc.at[idx[b]], sema,
                                           add=True))
            for b in range(ds):
                da[b].wait()
            return carry

        lax.fori_loop(0, groups, group, 0)
        plsc.subcore_barrier()
        pltpu.sync_copy(acc.at[pl.ds(s * rpt, rpt)],
                        out_hbm.at[pl.ds(c * npad + s * rpt, rpt)])

    k = pl.kernel(
        body,
        out_type=jax.ShapeDtypeStruct((_NC * npad, H), _F32),
        mesh=mesh,
        scratch_types=scratch,
    )
    return k(ea, row, zeros)


def kernel(x, edge_index, edge_attr, batch, params):
    n = x.shape[0]
    row, col = edge_index[0], edge_index[1]
    rpt, npad = _scatter_pad(n)
    zeros = jnp.zeros((rpt, H), _F32)

    ne_w1, ne_b1, ne_w2, ne_b2 = params["ne"]
    ee_w1, ee_b1, ee_w2, ee_b2 = params["ee"]
    nd_w1, nd_b1, nd_w2, nd_b2 = params["nd"]
    e0w1, e0b1, e0w2, e0b2 = params["enc0_edge"]
    e1w1, e1b1, e1w2, e1b2 = params["enc1_edge"]
    n0w1, n0b1, n0w2, n0b2 = params["enc0_node"]
    n1w1, n1b1, n1w2, n1b2 = params["enc1_node"]
    d0ew1, d0eb1, d0ew2, d0eb2 = params["dec0_edge"]
    d1ew1, d1eb1, d1ew2, d1eb2 = params["dec1_edge"]
    d0nw1, d0nb1, d0nw2, d0nb2 = params["dec0_node"]
    d1nw1, d1nb1, d1nw2, d1nb2 = params["dec1_node"]

    e0_r, e0_c, e0_e = e0w1[:H], e0w1[H:2 * H], e0w1[2 * H:]
    e1_r, e1_c, e1_e = e1w1[:H], e1w1[H:2 * H], e1w1[2 * H:]
    n0_x, n0_a = n0w1[:H], n0w1[H:]
    n1_x, n1_a = n1w1[:H], n1w1[H:]
    d0_xr, d0_er, d0_xc, d0_ec, d0_e = (d0ew1[:H], d0ew1[H:2 * H],
                                        d0ew1[2 * H:3 * H],
                                        d0ew1[3 * H:4 * H], d0ew1[4 * H:])
    d1_xr, d1_er, d1_xc, d1_ec, d1_e = (d1ew1[:H], d1ew1[H:2 * H],
                                        d1ew1[2 * H:3 * H],
                                        d1ew1[3 * H:4 * H], d1ew1[4 * H:])
    d0_nx, d0_ne, d0_na = d0nw1[:H], d0nw1[H:2 * H], d0nw1[2 * H:]
    d1_nx, d1_ne, d1_na = d1nw1[:H], d1nw1[H:2 * H], d1nw1[2 * H:]

    x1, u0, v0 = _mlp_sum([x], [ne_w1], ne_b1, ne_w2, ne_b2,
                          posts=[e0_r, e0_c])
    (t0,) = _mlp_sum([edge_attr], [ee_w1], ee_b1, ee_w2, ee_b2,
                     posts=[e0_e], emit_y=False, block_rows=2000)

    ug, vg = _sc_gather(u0, v0, row, col)
    ea_e0, t_e1, t_d1 = _edge_mlp(ug, vg, t0, e0b1, e0w2, e0b2,
                                  posts=[e1_e, d1_e])
    parts = _sc_scatter(ea_e0, row, zeros, n)
    p0, p1 = parts[:npad], parts[npad:]
    x2, u1, v1, ud1p, vd1p = _mlp_sum(
        [x1, p0, p1], [n0_x, n0_a, n0_a], n0b1, n0w2, n0b2,
        posts=[e1_r, e1_c, d1_er, d1_ec])

    ug, vg = _sc_gather(u1, v1, row, col)
    ea_e1, t_d0 = _edge_mlp(ug, vg, t_e1, e1b1, e1w2, e1b2, posts=[d0_e])
    parts = _sc_scatter(ea_e1, row, zeros, n)
    p0, p1 = parts[:npad], parts[npad:]
    x3, ud0, vd0 = _mlp_sum(
        [x2, p0, p1], [n1_x, n1_a, n1_a], n1b1, n1w2, n1b2,
        posts=[d0_xr + d0_er, d0_xc + d0_ec])

    ug, vg = _sc_gather(ud0, vd0, row, col)
    (ea_d0,) = _edge_mlp(ug, vg, t_d0, d0eb1, d0ew2, d0eb2)
    parts = _sc_scatter(ea_d0, row, zeros, n)
    p0, p1 = parts[:npad], parts[npad:]
    x4, ud1, vd1 = _mlp_sum(
        [x3, p0, p1], [d0_nx + d0_ne, d0_na, d0_na], d0nb1, d0nw2, d0nb2,
        posts=[d1_xr, d1_xc], post_adds=[ud1p, vd1p])

    ug, vg = _sc_gather(ud1, vd1, row, col)
    (ea_d1,) = _edge_mlp(ug, vg, t_d1, d1eb1, d1ew2, d1eb2)
    parts = _sc_scatter(ea_d1, row, zeros, n)
    p0, p1 = parts[:npad], parts[npad:]
    (x5,) = _mlp_sum([x4, x2, p0, p1], [d1_nx, d1_ne, d1_na, d1_na],
                     d1nb1, d1nw2, d1nb2)

    (out,) = _mlp_sum([x5], [nd_w1], nd_b1, nd_w2, nd_b2)
    return out


# speedup vs baseline: 3.5962x; 1.0523x over previous
import functools

import jax
import jax.numpy as jnp
from jax import lax
from jax.experimental import pallas as pl
from jax.experimental.pallas import tpu as pltpu
from jax.experimental.pallas import tpu_sc as plsc

H = 128
_F32 = jnp.float32
_BF16 = jnp.bfloat16


def _mlp_sum(inputs, w1s, b1, w2, b2, posts=(), post_packed=(),
             post_adds=None, emit_y=True, block_rows=1000):
    nin, npost = len(inputs), len(posts)
    if not post_packed:
        post_packed = [False] * npost
    if post_adds is None:
        post_adds = [None] * npost
    adds = [a for a in post_adds if a is not None]
    has_add = [a is not None for a in post_adds]
    rows = inputs[0].shape[0]
    assert rows % block_rows == 0
    grid = rows // block_rows

    def body(*refs):
        ins = refs[:nin]
        w1r = refs[nin:2 * nin]
        b1r, w2r, b2r = refs[2 * nin:2 * nin + 3]
        pw = refs[2 * nin + 3:2 * nin + 3 + npost]
        ad = refs[2 * nin + 3 + npost:2 * nin + 3 + npost + len(adds)]
        outs = refs[2 * nin + 3 + npost + len(adds):]
        acc = b1r[...].astype(_F32)
        for k in range(nin):
            acc = acc + jnp.dot(ins[k][...], w1r[k][...],
                                preferred_element_type=_F32)
        hid = jnp.maximum(acc, 0.0)
        y = jnp.dot(hid, w2r[...], preferred_element_type=_F32) + b2r[...]
        oi = 0
        if emit_y:
            outs[0][...] = y
            oi = 1
        ai = 0
        for j in range(npost):
            pv = jnp.dot(y, pw[j][...], preferred_element_type=_F32)
            if has_add[j]:
                pv = pv + ad[ai][...]
                ai += 1
            outs[oi + j][...] = pv.astype(_BF16) if post_packed[j] else pv

    in_specs = []
    for a in inputs:
        d = a.shape[1]
        in_specs.append(pl.BlockSpec((block_rows, d), lambda i: (i, 0)))
    for w in w1s:
        in_specs.append(pl.BlockSpec(w.shape, lambda i: (0, 0)))
    in_specs.append(pl.BlockSpec((1, H), lambda i: (0, 0)))
    in_specs.append(pl.BlockSpec(w2.shape, lambda i: (0, 0)))
    in_specs.append(pl.BlockSpec((1, H), lambda i: (0, 0)))
    for p in posts:
        in_specs.append(pl.BlockSpec(p.shape, lambda i: (0, 0)))
    for a in adds:
        in_specs.append(pl.BlockSpec((block_rows, H), lambda i: (i, 0)))

    out_shape = ([jax.ShapeDtypeStruct((rows, H), _F32)] if emit_y else [])
    out_specs = ([pl.BlockSpec((block_rows, H), lambda i: (i, 0))]
                 if emit_y else [])
    for j in range(npost):
        dt = _BF16 if post_packed[j] else _F32
        out_shape.append(jax.ShapeDtypeStruct((rows, H), dt))
        out_specs.append(pl.BlockSpec((block_rows, H), lambda i: (i, 0)))

    fn = pl.pallas_call(
        body,
        grid=(grid,),
        in_specs=in_specs,
        out_specs=out_specs,
        out_shape=out_shape,
    )
    args = (list(inputs) + list(w1s)
            + [b1.reshape(1, H), w2, b2.reshape(1, H)] + list(posts) + adds)
    return tuple(fn(*args))


def _edge_mlp(ug, vg, t, b1, w2, b2, posts=(), block_rows=2000):
    npost = len(posts)
    rows = ug.shape[0]
    assert rows % block_rows == 0
    grid = rows // block_rows

    def body(*refs):
        ugr, vgr, tr, b1r, w2r, b2r = refs[:6]
        pw = refs[6:6 + npost]
        outs = refs[6 + npost:]
        s = ugr[...] + vgr[...] + tr[...].astype(_F32) + b1r[...]
        hid = jnp.maximum(s, 0.0)
        ea = jnp.dot(hid, w2r[...], preferred_element_type=_F32) + b2r[...]
        outs[0][...] = ea
        for j in range(npost):
            outs[1 + j][...] = jnp.dot(
                ea, pw[j][...], preferred_element_type=_F32).astype(_BF16)

    in_specs = [pl.BlockSpec((block_rows, H), lambda i: (i, 0)),
                pl.BlockSpec((block_rows, H), lambda i: (i, 0)),
                pl.BlockSpec((block_rows, H), lambda i: (i, 0)),
                pl.BlockSpec((1, H), lambda i: (0, 0)),
                pl.BlockSpec((H, H), lambda i: (0, 0)),
                pl.BlockSpec((1, H), lambda i: (0, 0))]
    for p in posts:
        in_specs.append(pl.BlockSpec(p.shape, lambda i: (0, 0)))
    out_shape = [jax.ShapeDtypeStruct((rows, H), _F32)]
    out_specs = [pl.BlockSpec((block_rows, H), lambda i: (i, 0))]
    for _ in posts:
        out_shape.append(jax.ShapeDtypeStruct((rows, H), _BF16))
        out_specs.append(pl.BlockSpec((block_rows, H), lambda i: (i, 0)))
    fn = pl.pallas_call(body, grid=(grid,), in_specs=in_specs,
                        out_specs=out_specs, out_shape=out_shape)
    return tuple(fn(ug, vg, t, b1.reshape(1, H), w2, b2.reshape(1, H),
                    *posts))


_NC, _NS = 2, 16
_NW = _NC * _NS
_C = 80
_D = 5


def _sc_gather(u, v, row, col):
    e = row.shape[0]
    assert e % (_NW * _C * _D) == 0
    epw = e // _NW
    groups = epw // (_C * _D)
    mesh = plsc.VectorSubcoreMesh(core_axis_name="c", subcore_axis_name="s")
    scratch = ([pltpu.VMEM((_C,), jnp.int32) for _ in range(2 * _D)]
               + [pltpu.VMEM((_C, H), _F32) for _ in range(2 * _D)]
               + [pltpu.SemaphoreType.DMA] * 3)

    def body(u_hbm, v_hbm, row_hbm, col_hbm, ug_hbm, vg_hbm, *scr):
        idr = scr[0:_D]
        idc = scr[_D:2 * _D]
        bu = scr[2 * _D:3 * _D]
        bv = scr[3 * _D:4 * _D]
        semi, semg, semw = scr[4 * _D:]
        w = lax.axis_index("c") * _NS + lax.axis_index("s")
        base = w * epw

        def group(g, carry):
            off0 = base + g * (_C * _D)
            di = [(pltpu.async_copy(row_hbm.at[pl.ds(off0 + b * _C, _C)],
                                    idr[b], semi),
                   pltpu.async_copy(col_hbm.at[pl.ds(off0 + b * _C, _C)],
                                    idc[b], semi)) for b in range(_D)]
            dg = []
            for b in range(_D):
                di[b][0].wait()
                di[b][1].wait()
                dg.append((pltpu.async_copy(u_hbm.at[idr[b]], bu[b], semg),
                           pltpu.async_copy(v_hbm.at[idc[b]], bv[b], semg)))
            dw = []
            for b in range(_D):
                dg[b][0].wait()
                dg[b][1].wait()
                off = off0 + b * _C
                dw.append((pltpu.async_copy(bu[b], ug_hbm.at[pl.ds(off, _C)],
                                            semw),
                           pltpu.async_copy(bv[b], vg_hbm.at[pl.ds(off, _C)],
                                            semw)))
            for b in range(_D):
                dw[b][0].wait()
                dw[b][1].wait()
            return carry

        lax.fori_loop(0, groups, group, 0)

    k = pl.kernel(
        body,
        out_type=(jax.ShapeDtypeStruct((e, H), _F32),
                  jax.ShapeDtypeStruct((e, H), _F32)),
        mesh=mesh,
        scratch_types=scratch,
    )
    return k(u, v, row, col)


def _scatter_pad(n):
    rpt = ((n + _NS * 8 - 1) // (_NS * 8)) * 8
    return rpt, _NS * rpt


def _sc_scatter(ea, row, zeros, n):
    e = row.shape[0]
    rpt, npad = _scatter_pad(n)
    cs, ds = 40, 5
    assert e % (_NC * _NS * cs * ds) == 0
    epc = e // _NC
    ept = epc // _NS
    groups = ept // (cs * ds)
    mesh = plsc.VectorSubcoreMesh(core_axis_name="c", subcore_axis_name="s")
    scratch = ([pltpu.VMEM_SHARED((npad, H), _F32)]
               + [pltpu.VMEM((cs,), jnp.int32) for _ in range(ds)]
               + [pltpu.VMEM((cs, H), _F32) for _ in range(ds)]
               + [pltpu.SemaphoreType.DMA] * 2)

    def body(ea_hbm, row_hbm, z_hbm, out_hbm, *scr):
        acc = scr[0]
        idx = scr[1:1 + ds]
        buf = scr[1 + ds:1 + 2 * ds]
        semi, sema = scr[1 + 2 * ds:]
        c = lax.axis_index("c")
        s = lax.axis_index("s")
        pltpu.sync_copy(z_hbm, acc.at[pl.ds(s * rpt, rpt)])
        plsc.subcore_barrier()
        base = c * epc + s * ept

        def group(g, carry):
            off0 = base + g * (cs * ds)
            di = [(pltpu.async_copy(row_hbm.at[pl.ds(off0 + b * cs, cs)],
                                    idx[b], semi),
                   pltpu.async_copy(ea_hbm.at[pl.ds(off0 + b * cs, cs)],
                                    buf[b], semi)) for b in range(ds)]
            da = []
            for b in range(ds):
                di[b][0].wait()
                di[b][1].wait()
                da.append(pltpu.async_copy(buf[b], acc.at[idx[b]], sema,
                                           add=True))
            for b in range(ds):
                da[b].wait()
            return carry

        lax.fori_loop(0, groups, group, 0)
        plsc.subcore_barrier()
        pltpu.sync_copy(acc.at[pl.ds(s * rpt, rpt)],
                        out_hbm.at[pl.ds(c * npad + s * rpt, rpt)])

    k = pl.kernel(
        body,
        out_type=jax.ShapeDtypeStruct((_NC * npad, H), _F32),
        mesh=mesh,
        scratch_types=scratch,
    )
    return k(ea, row, zeros)


def kernel(x, edge_index, edge_attr, batch, params):
    n = x.shape[0]
    row, col = edge_index[0], edge_index[1]
    rpt, npad = _scatter_pad(n)
    zeros = jnp.zeros((rpt, H), _F32)

    ne_w1, ne_b1, ne_w2, ne_b2 = params["ne"]
    ee_w1, ee_b1, ee_w2, ee_b2 = params["ee"]
    nd_w1, nd_b1, nd_w2, nd_b2 = params["nd"]
    e0w1, e0b1, e0w2, e0b2 = params["enc0_edge"]
    e1w1, e1b1, e1w2, e1b2 = params["enc1_edge"]
    n0w1, n0b1, n0w2, n0b2 = params["enc0_node"]
    n1w1, n1b1, n1w2, n1b2 = params["enc1_node"]
    d0ew1, d0eb1, d0ew2, d0eb2 = params["dec0_edge"]
    d1ew1, d1eb1, d1ew2, d1eb2 = params["dec1_edge"]
    d0nw1, d0nb1, d0nw2, d0nb2 = params["dec0_node"]
    d1nw1, d1nb1, d1nw2, d1nb2 = params["dec1_node"]

    e0_r, e0_c, e0_e = e0w1[:H], e0w1[H:2 * H], e0w1[2 * H:]
    e1_r, e1_c, e1_e = e1w1[:H], e1w1[H:2 * H], e1w1[2 * H:]
    n0_x, n0_a = n0w1[:H], n0w1[H:]
    n1_x, n1_a = n1w1[:H], n1w1[H:]
    d0_xr, d0_er, d0_xc, d0_ec, d0_e = (d0ew1[:H], d0ew1[H:2 * H],
                                        d0ew1[2 * H:3 * H],
                                        d0ew1[3 * H:4 * H], d0ew1[4 * H:])
    d1_xr, d1_er, d1_xc, d1_ec, d1_e = (d1ew1[:H], d1ew1[H:2 * H],
                                        d1ew1[2 * H:3 * H],
                                        d1ew1[3 * H:4 * H], d1ew1[4 * H:])
    d0_nx, d0_ne, d0_na = d0nw1[:H], d0nw1[H:2 * H], d0nw1[2 * H:]
    d1_nx, d1_ne, d1_na = d1nw1[:H], d1nw1[H:2 * H], d1nw1[2 * H:]

    x1, u0, v0 = _mlp_sum([x], [ne_w1], ne_b1, ne_w2, ne_b2,
                          posts=[e0_r, e0_c])
    (t0,) = _mlp_sum([edge_attr], [ee_w1], ee_b1, ee_w2, ee_b2,
                     posts=[e0_e], post_packed=[True], emit_y=False,
                     block_rows=2000)

    ug, vg = _sc_gather(u0, v0, row, col)
    ea_e0, t_e1, t_d1 = _edge_mlp(ug, vg, t0, e0b1, e0w2, e0b2,
                                  posts=[e1_e, d1_e])
    parts = _sc_scatter(ea_e0, row, zeros, n)
    p0, p1 = parts[:npad], parts[npad:]
    x2, u1, v1, ud1p, vd1p = _mlp_sum(
        [x1, p0, p1], [n0_x, n0_a, n0_a], n0b1, n0w2, n0b2,
        posts=[e1_r, e1_c, d1_er, d1_ec])

    ug, vg = _sc_gather(u1, v1, row, col)
    ea_e1, t_d0 = _edge_mlp(ug, vg, t_e1, e1b1, e1w2, e1b2, posts=[d0_e])
    parts = _sc_scatter(ea_e1, row, zeros, n)
    p0, p1 = parts[:npad], parts[npad:]
    x3, ud0, vd0 = _mlp_sum(
        [x2, p0, p1], [n1_x, n1_a, n1_a], n1b1, n1w2, n1b2,
        posts=[d0_xr + d0_er, d0_xc + d0_ec])

    ug, vg = _sc_gather(ud0, vd0, row, col)
    (ea_d0,) = _edge_mlp(ug, vg, t_d0, d0eb1, d0ew2, d0eb2)
    parts = _sc_scatter(ea_d0, row, zeros, n)
    p0, p1 = parts[:npad], parts[npad:]
    x4, ud1, vd1 = _mlp_sum(
        [x3, p0, p1], [d0_nx + d0_ne, d0_na, d0_na], d0nb1, d0nw2, d0nb2,
        posts=[d1_xr, d1_xc], post_adds=[ud1p, vd1p])

    ug, vg = _sc_gather(ud1, vd1, row, col)
    (ea_d1,) = _edge_mlp(ug, vg, t_d1, d1eb1, d1ew2, d1eb2)
    parts = _sc_scatter(ea_d1, row, zeros, n)
    p0, p1 = parts[:npad], parts[npad:]
    (x5,) = _mlp_sum([x4, x2, p0, p1], [d1_nx, d1_ne, d1_na, d1_na],
                     d1nb1, d1nw2, d1nb2)

    (out,) = _mlp_sum([x5], [nd_w1], nd_b1, nd_w2, nd_b2)
    return out


# speedup vs baseline: 3.6766x; 1.0224x over previous
import functools

import jax
import jax.numpy as jnp
from jax import lax
from jax.experimental import pallas as pl
from jax.experimental.pallas import tpu as pltpu
from jax.experimental.pallas import tpu_sc as plsc

H = 128
_F32 = jnp.float32
_BF16 = jnp.bfloat16


def _mlp_sum(inputs, w1s, b1, w2, b2, posts=(), post_packed=(),
             post_adds=None, emit_y=True, block_rows=1000):
    nin, npost = len(inputs), len(posts)
    if not post_packed:
        post_packed = [False] * npost
    if post_adds is None:
        post_adds = [None] * npost
    adds = [a for a in post_adds if a is not None]
    has_add = [a is not None for a in post_adds]
    rows = inputs[0].shape[0]
    assert rows % block_rows == 0
    grid = rows // block_rows

    def body(*refs):
        ins = refs[:nin]
        w1r = refs[nin:2 * nin]
        b1r, w2r, b2r = refs[2 * nin:2 * nin + 3]
        pw = refs[2 * nin + 3:2 * nin + 3 + npost]
        ad = refs[2 * nin + 3 + npost:2 * nin + 3 + npost + len(adds)]
        outs = refs[2 * nin + 3 + npost + len(adds):]
        acc = b1r[...].astype(_F32)
        for k in range(nin):
            acc = acc + jnp.dot(ins[k][...], w1r[k][...],
                                preferred_element_type=_F32)
        hid = jnp.maximum(acc, 0.0)
        y = jnp.dot(hid, w2r[...], preferred_element_type=_F32) + b2r[...]
        oi = 0
        if emit_y:
            outs[0][...] = y
            oi = 1
        ai = 0
        for j in range(npost):
            pv = jnp.dot(y, pw[j][...], preferred_element_type=_F32)
            if has_add[j]:
                pv = pv + ad[ai][...]
                ai += 1
            outs[oi + j][...] = pv.astype(_BF16) if post_packed[j] else pv

    in_specs = []
    for a in inputs:
        d = a.shape[1]
        in_specs.append(pl.BlockSpec((block_rows, d), lambda i: (i, 0)))
    for w in w1s:
        in_specs.append(pl.BlockSpec(w.shape, lambda i: (0, 0)))
    in_specs.append(pl.BlockSpec((1, H), lambda i: (0, 0)))
    in_specs.append(pl.BlockSpec(w2.shape, lambda i: (0, 0)))
    in_specs.append(pl.BlockSpec((1, H), lambda i: (0, 0)))
    for p in posts:
        in_specs.append(pl.BlockSpec(p.shape, lambda i: (0, 0)))
    for a in adds:
        in_specs.append(pl.BlockSpec((block_rows, H), lambda i: (i, 0)))

    out_shape = ([jax.ShapeDtypeStruct((rows, H), _F32)] if emit_y else [])
    out_specs = ([pl.BlockSpec((block_rows, H), lambda i: (i, 0))]
                 if emit_y else [])
    for j in range(npost):
        dt = _BF16 if post_packed[j] else _F32
        out_shape.append(jax.ShapeDtypeStruct((rows, H), dt))
        out_specs.append(pl.BlockSpec((block_rows, H), lambda i: (i, 0)))

    fn = pl.pallas_call(
        body,
        grid=(grid,),
        in_specs=in_specs,
        out_specs=out_specs,
        out_shape=out_shape,
    )
    args = (list(inputs) + list(w1s)
            + [b1.reshape(1, H), w2, b2.reshape(1, H)] + list(posts) + adds)
    return tuple(fn(*args))


def _edge_mlp(ug, vg, t, b1, w2, b2, posts=(), block_rows=2000):
    npost = len(posts)
    rows = ug.shape[0]
    assert rows % block_rows == 0
    grid = rows // block_rows

    def body(*refs):
        ugr, vgr, tr, b1r, w2r, b2r = refs[:6]
        pw = refs[6:6 + npost]
        outs = refs[6 + npost:]
        s = ugr[...] + vgr[...] + tr[...].astype(_F32) + b1r[...]
        hid = jnp.maximum(s, 0.0)
        ea = jnp.dot(hid, w2r[...], preferred_element_type=_F32) + b2r[...]
        outs[0][...] = ea
        for j in range(npost):
            outs[1 + j][...] = jnp.dot(
                ea, pw[j][...], preferred_element_type=_F32).astype(_BF16)

    in_specs = [pl.BlockSpec((block_rows, H), lambda i: (i, 0)),
                pl.BlockSpec((block_rows, H), lambda i: (i, 0)),
                pl.BlockSpec((block_rows, H), lambda i: (i, 0)),
                pl.BlockSpec((1, H), lambda i: (0, 0)),
                pl.BlockSpec((H, H), lambda i: (0, 0)),
                pl.BlockSpec((1, H), lambda i: (0, 0))]
    for p in posts:
        in_specs.append(pl.BlockSpec(p.shape, lambda i: (0, 0)))
    out_shape = [jax.ShapeDtypeStruct((rows, H), _F32)]
    out_specs = [pl.BlockSpec((block_rows, H), lambda i: (i, 0))]
    for _ in posts:
        out_shape.append(jax.ShapeDtypeStruct((rows, H), _BF16))
        out_specs.append(pl.BlockSpec((block_rows, H), lambda i: (i, 0)))
    fn = pl.pallas_call(body, grid=(grid,), in_specs=in_specs,
                        out_specs=out_specs, out_shape=out_shape)
    return tuple(fn(ug, vg, t, b1.reshape(1, H), w2, b2.reshape(1, H),
                    *posts))


_NC, _NS = 2, 16
_NW = _NC * _NS
_C = 40
_D = 5


def _sc_gather(u, v, row, col):
    e = row.shape[0]
    assert e % (_NW * _C * _D) == 0
    epw = e // _NW
    groups = epw // (_C * _D)
    mesh = plsc.VectorSubcoreMesh(core_axis_name="c", subcore_axis_name="s")
    scratch = ([pltpu.VMEM((_C,), jnp.int32) for _ in range(2 * _D)]
               + [pltpu.VMEM((_C, H), _F32) for _ in range(2 * _D)]
               + [pltpu.SemaphoreType.DMA] * 3)

    def body(u_hbm, v_hbm, row_hbm, col_hbm, ug_hbm, vg_hbm, *scr):
        idr = scr[0:_D]
        idc = scr[_D:2 * _D]
        bu = scr[2 * _D:3 * _D]
        bv = scr[3 * _D:4 * _D]
        semi, semg, semw = scr[4 * _D:]
        w = lax.axis_index("c") * _NS + lax.axis_index("s")
        base = w * epw

        def group(g, carry):
            off0 = base + g * (_C * _D)
            di = [(pltpu.async_copy(row_hbm.at[pl.ds(off0 + b * _C, _C)],
                                    idr[b], semi),
                   pltpu.async_copy(col_hbm.at[pl.ds(off0 + b * _C, _C)],
                                    idc[b], semi)) for b in range(_D)]
            dg = []
            for b in range(_D):
                di[b][0].wait()
                di[b][1].wait()
                dg.append((pltpu.async_copy(u_hbm.at[idr[b]], bu[b], semg),
                           pltpu.async_copy(v_hbm.at[idc[b]], bv[b], semg)))
            dw = []
            for b in range(_D):
                dg[b][0].wait()
                dg[b][1].wait()
                off = off0 + b * _C
                dw.append((pltpu.async_copy(bu[b], ug_hbm.at[pl.ds(off, _C)],
                                            semw),
                           pltpu.async_copy(bv[b], vg_hbm.at[pl.ds(off, _C)],
                                            semw)))
            for b in range(_D):
                dw[b][0].wait()
                dw[b][1].wait()
            return carry

        lax.fori_loop(0, groups, group, 0)

    k = pl.kernel(
        body,
        out_type=(jax.ShapeDtypeStruct((e, H), _F32),
                  jax.ShapeDtypeStruct((e, H), _F32)),
        mesh=mesh,
        scratch_types=scratch,
    )
    return k(u, v, row, col)


def _scatter_pad(n):
    rpt = ((n + _NS * 8 - 1) // (_NS * 8)) * 8
    return rpt, _NS * rpt


def _sc_scatter(ea, row, zeros, n):
    e = row.shape[0]
    rpt, npad = _scatter_pad(n)
    cs, ds = 40, 5
    assert e % (_NC * _NS * cs * ds) == 0
    epc = e // _NC
    ept = epc // _NS
    groups = ept // (cs * ds)
    mesh = plsc.VectorSubcoreMesh(core_axis_name="c", subcore_axis_name="s")
    scratch = ([pltpu.VMEM_SHARED((npad, H), _F32)]
               + [pltpu.VMEM((cs,), jnp.int32) for _ in range(ds)]
               + [pltpu.VMEM((cs, H), _F32) for _ in range(ds)]
               + [pltpu.SemaphoreType.DMA] * 2)

    def body(ea_hbm, row_hbm, z_hbm, out_hbm, *scr):
        acc = scr[0]
        idx = scr[1:1 + ds]
        buf = scr[1 + ds:1 + 2 * ds]
        semi, sema = scr[1 + 2 * ds:]
        c = lax.axis_index("c")
        s = lax.axis_index("s")
        pltpu.sync_copy(z_hbm, acc.at[pl.ds(s * rpt, rpt)])
        plsc.subcore_barrier()
        base = c * epc + s * ept

        def group(g, carry):
            off0 = base + g * (cs * ds)
            di = [(pltpu.async_copy(row_hbm.at[pl.ds(off0 + b * cs, cs)],
                                    idx[b], semi),
                   pltpu.async_copy(ea_hbm.at[pl.ds(off0 + b * cs, cs)],
                                    buf[b], semi)) for b in range(ds)]
            da = []
            for b in range(ds):
                di[b][0].wait()
                di[b][1].wait()
                da.append(pltpu.async_copy(buf[b], acc.at[idx[b]], sema,
                                           add=True))
            for b in range(ds):
                da[b].wait()
            return carry

        lax.fori_loop(0, groups, group, 0)
        plsc.subcore_barrier()
        pltpu.sync_copy(acc.at[pl.ds(s * rpt, rpt)],
                        out_hbm.at[pl.ds(c * npad + s * rpt, rpt)])

    k = pl.kernel(
        body,
        out_type=jax.ShapeDtypeStruct((_NC * npad, H), _F32),
        mesh=mesh,
        scratch_types=scratch,
    )
    return k(ea, row, zeros)


def kernel(x, edge_index, edge_attr, batch, params):
    n = x.shape[0]
    row, col = edge_index[0], edge_index[1]
    rpt, npad = _scatter_pad(n)
    zeros = jnp.zeros((rpt, H), _F32)

    ne_w1, ne_b1, ne_w2, ne_b2 = params["ne"]
    ee_w1, ee_b1, ee_w2, ee_b2 = params["ee"]
    nd_w1, nd_b1, nd_w2, nd_b2 = params["nd"]
    e0w1, e0b1, e0w2, e0b2 = params["enc0_edge"]
    e1w1, e1b1, e1w2, e1b2 = params["enc1_edge"]
    n0w1, n0b1, n0w2, n0b2 = params["enc0_node"]
    n1w1, n1b1, n1w2, n1b2 = params["enc1_node"]
    d0ew1, d0eb1, d0ew2, d0eb2 = params["dec0_edge"]
    d1ew1, d1eb1, d1ew2, d1eb2 = params["dec1_edge"]
    d0nw1, d0nb1, d0nw2, d0nb2 = params["dec0_node"]
    d1nw1, d1nb1, d1nw2, d1nb2 = params["dec1_node"]

    e0_r, e0_c, e0_e = e0w1[:H], e0w1[H:2 * H], e0w1[2 * H:]
    e1_r, e1_c, e1_e = e1w1[:H], e1w1[H:2 * H], e1w1[2 * H:]
    n0_x, n0_a = n0w1[:H], n0w1[H:]
    n1_x, n1_a = n1w1[:H], n1w1[H:]
    d0_xr, d0_er, d0_xc, d0_ec, d0_e = (d0ew1[:H], d0ew1[H:2 * H],
                                        d0ew1[2 * H:3 * H],
                                        d0ew1[3 * H:4 * H], d0ew1[4 * H:])
    d1_xr, d1_er, d1_xc, d1_ec, d1_e = (d1ew1[:H], d1ew1[H:2 * H],
                                        d1ew1[2 * H:3 * H],
                                        d1ew1[3 * H:4 * H], d1ew1[4 * H:])
    d0_nx, d0_ne, d0_na = d0nw1[:H], d0nw1[H:2 * H], d0nw1[2 * H:]
    d1_nx, d1_ne, d1_na = d1nw1[:H], d1nw1[H:2 * H], d1nw1[2 * H:]

    e = row.shape[0]
    eh = e // 2
    rh = (row[:eh], row[eh:])
    ch = (col[:eh], col[eh:])

    def mp_layer(u, v, ts, eb1, ew2, eb2, eposts):
        g1 = _sc_gather(u, v, rh[0], ch[0])
        g2 = _sc_gather(u, v, rh[1], ch[1])
        o1 = _edge_mlp(g1[0], g1[1], ts[0], eb1, ew2, eb2, posts=eposts)
        s1 = _sc_scatter(o1[0], rh[0], zeros, n)
        o2 = _edge_mlp(g2[0], g2[1], ts[1], eb1, ew2, eb2, posts=eposts)
        s2 = _sc_scatter(o2[0], rh[1], zeros, n)
        parts = [s1[:npad], s1[npad:], s2[:npad], s2[npad:]]
        return (o1[1:], o2[1:]), parts

    x1, u0, v0 = _mlp_sum([x], [ne_w1], ne_b1, ne_w2, ne_b2,
                          posts=[e0_r, e0_c])
    t0 = tuple(_mlp_sum([ea_h], [ee_w1], ee_b1, ee_w2, ee_b2,
                        posts=[e0_e], post_packed=[True], emit_y=False,
                        block_rows=2000)[0]
               for ea_h in (edge_attr[:eh], edge_attr[eh:]))

    (tp1, tp2), parts = mp_layer(u0, v0, t0, e0b1, e0w2, e0b2,
                                 [e1_e, d1_e])
    t_e1 = (tp1[0], tp2[0])
    t_d1 = (tp1[1], tp2[1])
    x2, u1, v1, ud1p, vd1p = _mlp_sum(
        [x1] + parts, [n0_x, n0_a, n0_a, n0_a, n0_a], n0b1, n0w2, n0b2,
        posts=[e1_r, e1_c, d1_er, d1_ec])

    (tp1, tp2), parts = mp_layer(u1, v1, t_e1, e1b1, e1w2, e1b2, [d0_e])
    t_d0 = (tp1[0], tp2[0])
    x3, ud0, vd0 = _mlp_sum(
        [x2] + parts, [n1_x, n1_a, n1_a, n1_a, n1_a], n1b1, n1w2, n1b2,
        posts=[d0_xr + d0_er, d0_xc + d0_ec])

    _, parts = mp_layer(ud0, vd0, t_d0, d0eb1, d0ew2, d0eb2, [])
    x4, ud1, vd1 = _mlp_sum(
        [x3] + parts, [d0_nx + d0_ne, d0_na, d0_na, d0_na, d0_na],
        d0nb1, d0nw2, d0nb2,
        posts=[d1_xr, d1_xc], post_adds=[ud1p, vd1p])

    _, parts = mp_layer(ud1, vd1, t_d1, d1eb1, d1ew2, d1eb2, [])
    (x5,) = _mlp_sum([x4, x2] + parts,
                     [d1_nx, d1_ne, d1_na, d1_na, d1_na, d1_na],
                     d1nb1, d1nw2, d1nb2)

    (out,) = _mlp_sum([x5], [nd_w1], nd_b1, nd_w2, nd_b2)
    return out
